# speedup vs baseline: 9.1532x; 9.1532x over previous
import functools

import jax
import jax.numpy as jnp
from jax import lax
from jax.experimental import pallas as pl
from jax.experimental.pallas import tpu as pltpu
from jax.experimental.pallas import tpu_sc as plsc

_N = 10000
_E = 320000
_D = 128
_B = 8
_NP = 10240
_RB = 128
_NBLK = _NP // _RB
_NC = 2
_NS = 16
_NW = _NC * _NS
_CH = 128
_NCH = 79
_EPW = _CH * _NCH
_EP = _EPW * _NW
_RPT = _NP // _NS

_mesh = plsc.VectorSubcoreMesh(core_axis_name="c", subcore_axis_name="s",
                               num_cores=_NC, num_subcores=_NS)


def _sc_deg_body(dst_hbm, ones_hbm, z_hbm, deg_hbm, ones_v, idx_v, acc_sh):
    c = lax.axis_index("c")
    s = lax.axis_index("s")
    wid = c * _NS + s
    pltpu.sync_copy(ones_hbm, ones_v)
    pltpu.sync_copy(z_hbm, acc_sh.at[pl.ds(s * _RPT, _RPT)])
    plsc.subcore_barrier()

    def step(i, carry):
        base = wid * _EPW + i * _CH
        pltpu.sync_copy(dst_hbm.at[pl.ds(base, _CH)], idx_v)
        pltpu.sync_copy(ones_v, acc_sh.at[idx_v], add=True)
        return carry

    lax.fori_loop(0, _NCH, step, 0)
    plsc.subcore_barrier()
    pltpu.sync_copy(acc_sh.at[pl.ds(s * _RPT, _RPT)],
                    deg_hbm.at[c, pl.ds(s * _RPT, _RPT)])


_sc_deg = pl.kernel(
    _sc_deg_body,
    out_type=jax.ShapeDtypeStruct((_NC, _NP, 16), jnp.float32),
    mesh=_mesh,
    scratch_types=[
        pltpu.VMEM((_CH, 16), jnp.float32),
        pltpu.VMEM((_CH,), jnp.int32),
        pltpu.VMEM_SHARED((_NP, 16), jnp.float32),
    ],
)


def _sc_scat_body(h_hbm, src_hbm, dst_hbm, z_hbm, acc_hbm,
                  idx_s, idx_d, rows_v, acc_sh, sem):
    c = lax.axis_index("c")
    s = lax.axis_index("s")
    wid = c * _NS + s
    pltpu.sync_copy(z_hbm, acc_sh.at[pl.ds(s * _RPT, _RPT)])
    plsc.subcore_barrier()

    def step(i, carry):
        base = wid * _EPW + i * _CH
        pltpu.sync_copy(src_hbm.at[pl.ds(base, _CH)], idx_s)
        pltpu.sync_copy(dst_hbm.at[pl.ds(base, _CH)], idx_d)
        pltpu.async_copy(h_hbm.at[idx_s], rows_v, sem).wait()
        pltpu.sync_copy(rows_v, acc_sh.at[idx_d], add=True)
        return carry

    lax.fori_loop(0, _NCH, step, 0)
    plsc.subcore_barrier()
    pltpu.sync_copy(acc_sh.at[pl.ds(s * _RPT, _RPT)],
                    acc_hbm.at[c, pl.ds(s * _RPT, _RPT)])


_sc_scat = pl.kernel(
    _sc_scat_body,
    out_type=jax.ShapeDtypeStruct((_NC, _NP, _D), jnp.float32),
    mesh=_mesh,
    scratch_types=[
        pltpu.VMEM((_CH,), jnp.int32),
        pltpu.VMEM((_CH,), jnp.int32),
        pltpu.VMEM((_CH, _D), jnp.float32),
        pltpu.VMEM_SHARED((_NP, _D), jnp.float32),
        pltpu.SemaphoreType.DMA,
    ],
)


def _tc1_body(x_ref, w_ref, d0_ref, d1_ref, h_ref, dv_ref):
    deg = d0_ref[:, 0:1] + d1_ref[:, 0:1] + 1.0
    dinv = lax.rsqrt(deg)
    h = jnp.dot(x_ref[...], w_ref[...], preferred_element_type=jnp.float32)
    h_ref[...] = h * dinv
    dv_ref[...] = jnp.broadcast_to(dinv, (_RB, _D))


_tc1 = pl.pallas_call(
    _tc1_body,
    grid=(_NBLK,),
    in_specs=[
        pl.BlockSpec((_RB, _D), lambda i: (i, 0)),
        pl.BlockSpec((_D, _D), lambda i: (0, 0)),
        pl.BlockSpec((_RB, 16), lambda i: (i, 0)),
        pl.BlockSpec((_RB, 16), lambda i: (i, 0)),
    ],
    out_specs=[pl.BlockSpec((_RB, _D), lambda i: (i, 0))] * 2,
    out_shape=[jax.ShapeDtypeStruct((_NP, _D), jnp.float32)] * 2,
)


def _tc2_body(a0_ref, a1_ref, hp_ref, dv_ref, b_ref, w_ref, out_ref):
    pre = (a0_ref[...] + a1_ref[...] + hp_ref[...]) * dv_ref[...] + b_ref[...]
    g = jnp.maximum(pre, 0.0)
    h2 = jnp.dot(g, w_ref[...], preferred_element_type=jnp.float32)
    out_ref[...] = h2 * dv_ref[...]


_tc2 = pl.pallas_call(
    _tc2_body,
    grid=(_NBLK,),
    in_specs=[
        pl.BlockSpec((_RB, _D), lambda i: (i, 0)),
        pl.BlockSpec((_RB, _D), lambda i: (i, 0)),
        pl.BlockSpec((_RB, _D), lambda i: (i, 0)),
        pl.BlockSpec((_RB, _D), lambda i: (i, 0)),
        pl.BlockSpec((1, _D), lambda i: (0, 0)),
        pl.BlockSpec((_D, _D), lambda i: (0, 0)),
    ],
    out_specs=pl.BlockSpec((_RB, _D), lambda i: (i, 0)),
    out_shape=jax.ShapeDtypeStruct((_NP, _D), jnp.float32),
)


def _tc3_body(a0_ref, a1_ref, hp_ref, dv_ref, b_ref, bm_ref, wfc_ref,
              wex_ref, bfc_ref, af_ref, out_ref, pool_scr, cnt_scr):
    i = pl.program_id(0)

    @pl.when(i == 0)
    def _():
        pool_scr[...] = jnp.zeros_like(pool_scr)
        cnt_scr[...] = jnp.zeros_like(cnt_scr)

    pre = (a0_ref[...] + a1_ref[...] + hp_ref[...]) * dv_ref[...] + b_ref[...]
    g = jnp.maximum(pre, 0.0)
    lane = lax.broadcasted_iota(jnp.int32, (_RB, _D), 1)
    m = jnp.where((bm_ref[...] == lane) & (lane < _B), 1.0, 0.0)
    dn = (((0,), (0,)), ((), ()))
    pool_scr[...] += lax.dot_general(m, g, dn,
                                     preferred_element_type=jnp.float32)
    cnt_scr[...] += lax.dot_general(m, jnp.ones_like(g), dn,
                                    preferred_element_type=jnp.float32)

    @pl.when(i == _NBLK - 1)
    def _():
        pooled = pool_scr[...] / jnp.maximum(cnt_scr[...], 1.0)
        t = pooled * wfc_ref[...]
        ssum = jnp.sum(t, axis=1, keepdims=True)
        out_ref[...] = ssum + af_ref[...] * wex_ref[...] + bfc_ref[...]


_tc3 = pl.pallas_call(
    _tc3_body,
    grid=(_NBLK,),
    in_specs=[
        pl.BlockSpec((_RB, _D), lambda i: (i, 0)),
        pl.BlockSpec((_RB, _D), lambda i: (i, 0)),
        pl.BlockSpec((_RB, _D), lambda i: (i, 0)),
        pl.BlockSpec((_RB, _D), lambda i: (i, 0)),
        pl.BlockSpec((1, _D), lambda i: (0, 0)),
        pl.BlockSpec((_RB, _D), lambda i: (i, 0)),
        pl.BlockSpec((1, _D), lambda i: (0, 0)),
        pl.BlockSpec((1, _D), lambda i: (0, 0)),
        pl.BlockSpec((1, _D), lambda i: (0, 0)),
        pl.BlockSpec((_RB, _D), lambda i: (0, 0)),
    ],
    out_specs=pl.BlockSpec((_RB, _D), lambda i: (0, 0)),
    out_shape=jax.ShapeDtypeStruct((_RB, _D), jnp.float32),
    compiler_params=pltpu.CompilerParams(
        dimension_semantics=("arbitrary",)),
    scratch_shapes=[
        pltpu.VMEM((_RB, _D), jnp.float32),
        pltpu.VMEM((_RB, _D), jnp.float32),
    ],
)


@jax.jit
def kernel(x, edge_index, batch, additional_features, W1, b1, W2, b2, Wfc, bfc):
    f32 = jnp.float32
    padv = jnp.full((_EP - _E,), _N, jnp.int32)
    src_p = jnp.concatenate([edge_index[0].astype(jnp.int32), padv])
    dst_p = jnp.concatenate([edge_index[1].astype(jnp.int32), padv])
    x_p = jnp.zeros((_NP, _D), f32).at[:_N].set(x)

    ones_ch = jnp.ones((_CH, 16), f32)
    zeros_deg = jnp.zeros((_RPT, 16), f32)
    zeros_rows = jnp.zeros((_RPT, _D), f32)

    deg = _sc_deg(dst_p, ones_ch, zeros_deg)
    h1p, dvb = _tc1(x_p, W1, deg[0], deg[1])
    acc1 = _sc_scat(h1p, src_p, dst_p, zeros_rows)
    h2p = _tc2(acc1[0], acc1[1], h1p, dvb, b1.reshape(1, _D), W2)
    acc2 = _sc_scat(h2p, src_p, dst_p, zeros_rows)

    batch_p = jnp.concatenate(
        [batch.astype(jnp.int32), jnp.full((_NP - _N,), _B, jnp.int32)])
    batch_b = jnp.broadcast_to(batch_p[:, None], (_NP, _D))
    wfc_row = Wfc[:_D, 0].reshape(1, _D)
    wex = jnp.broadcast_to(Wfc[_D, 0].reshape(1, 1), (1, _D))
    bfc_b = jnp.broadcast_to(bfc.reshape(1, 1), (1, _D))
    af = jnp.zeros((_RB, _D), f32).at[:_B].set(
        jnp.broadcast_to(additional_features, (_B, _D)))

    outf = _tc3(acc2[0], acc2[1], h2p, dvb, b2.reshape(1, _D),
                batch_b, wfc_row, wex, bfc_b, af)
    return outf[:_B, :1]
